# baseline (speedup 1.0000x reference)
import functools

import jax
import jax.numpy as jnp
from jax import lax
from jax.experimental import pallas as pl
from jax.experimental.pallas import tpu as pltpu
from jax.experimental.pallas import tpu_sc as plsc

B = 16384
D = 128
NC = 2
NS = 16
NW = NC * NS
BPW = B // NW
CH = 128
NCH = BPW // CH
L = 16


def _body(table_h, idx1_h, idx2_h, w_h, bv_h, out_h,
          idx1_v, idx2_v, rows1_v, rows2_v, w_v, bv_v, tbuf, out_v,
          sem1, sem2):
    c = lax.axis_index("c")
    s = lax.axis_index("s")
    wid = s * NC + c

    pltpu.sync_copy(idx1_h.at[wid], idx1_v)
    pltpu.sync_copy(idx2_h.at[wid], idx2_v)
    pltpu.sync_copy(w_h, w_v)
    pltpu.sync_copy(bv_h, bv_v)

    wv = [w_v[pl.ds(k * L, L)] for k in range(2 * D // L)]
    bv = bv_v[...]
    iota16 = lax.iota(jnp.int32, L) * L

    def start(ci, buf):
        pltpu.async_copy(
            table_h.at[idx1_v.at[ci]], rows1_v.at[buf], sem1.at[buf])
        pltpu.async_copy(
            table_h.at[idx2_v.at[ci]], rows2_v.at[buf], sem2.at[buf])

    def drain(buf):
        dummy = table_h.at[pl.ds(0, CH)]
        pltpu.make_async_copy(dummy, rows1_v.at[buf], sem1.at[buf]).wait()
        pltpu.make_async_copy(dummy, rows2_v.at[buf], sem2.at[buf]).wait()

    start(0, 0)

    def chunk(ci, _):
        buf = lax.rem(ci, 2)

        @pl.when(ci + 1 < NCH)
        def _():
            start(ci + 1, 1 - buf)

        drain(buf)

        def group(g, _):
            base = g * L

            def sub(j, _):
                r0 = base + j * 4
                for i in range(4):
                    r = r0 + i
                    acc = wv[0] * rows1_v[buf, r, pl.ds(0, L)]
                    for k in range(1, D // L):
                        acc = acc + wv[k] * rows1_v[buf, r, pl.ds(k * L, L)]
                    for k in range(D // L):
                        acc = acc + wv[D // L + k] * rows2_v[buf, r,
                                                            pl.ds(k * L, L)]
                    plsc.store_scatter(tbuf, [iota16 + (j * 4 + i)], acc)
                return 0

            lax.fori_loop(0, 4, sub, 0)
            tot = tbuf[pl.ds(0, L)]
            for l in range(1, L):
                tot = tot + tbuf[pl.ds(l * L, L)]
            y = 1.0 / (1.0 + jnp.exp(-(tot + bv)))
            out_v[pl.ds(ci * CH + base, L)] = y
            return 0

        lax.fori_loop(0, CH // L, group, 0)
        return 0

    lax.fori_loop(0, NCH, chunk, 0)

    pltpu.sync_copy(out_v, out_h.at[pl.ds(wid * BPW, BPW)])


@jax.jit
def _run(table, idx1, idx2, w, bv):
    mesh = plsc.VectorSubcoreMesh(core_axis_name="c", subcore_axis_name="s")
    k = functools.partial(
        pl.kernel,
        mesh=mesh,
        compiler_params=pltpu.CompilerParams(needs_layout_passes=False),
        out_type=jax.ShapeDtypeStruct((B,), jnp.float32),
        scratch_types=[
            pltpu.VMEM((NCH, CH), jnp.int32),
            pltpu.VMEM((NCH, CH), jnp.int32),
            pltpu.VMEM((2, CH, D), jnp.float32),
            pltpu.VMEM((2, CH, D), jnp.float32),
            pltpu.VMEM((2 * D,), jnp.float32),
            pltpu.VMEM((L,), jnp.float32),
            pltpu.VMEM((L * L,), jnp.float32),
            pltpu.VMEM((BPW,), jnp.float32),
            pltpu.SemaphoreType.DMA((2,)),
            pltpu.SemaphoreType.DMA((2,)),
        ],
    )(_body)
    return k(table, idx1, idx2, w, bv)


def kernel(input1, input2, emb_table, fc_w, fc_b):
    idx1 = input1.astype(jnp.int32).reshape(NW, NCH, CH)
    idx2 = input2.astype(jnp.int32).reshape(NW, NCH, CH)
    w = fc_w.reshape(2 * D)
    bv = jnp.broadcast_to(fc_b.reshape(()), (L,)).astype(jnp.float32)
    out = _run(emb_table, idx1, idx2, w, bv)
    return out.reshape(B, 1)


# speedup vs baseline: 1.1840x; 1.1840x over previous
import functools

import jax
import jax.numpy as jnp
from jax import lax
from jax.experimental import pallas as pl
from jax.experimental.pallas import tpu as pltpu
from jax.experimental.pallas import tpu_sc as plsc

B = 16384
D = 128
NC = 2
NS = 16
NW = NC * NS
BPW = B // NW
CH = 128
NCH = BPW // CH
L = 16
NBUF = 3


def _body(table_h, idx_h, wb_h, out_h,
          idx_v, rows1_v, rows2_v, wb_v, out_v,
          sem1, sem2):
    c = lax.axis_index("c")
    s = lax.axis_index("s")
    wid = s * NC + c

    pltpu.sync_copy(idx_h.at[wid], idx_v)
    pltpu.sync_copy(wb_h, wb_v)

    wv = [wb_v[pl.ds(k * L, L)] for k in range(2 * D // L)]
    bv = wb_v[pl.ds(2 * D, L)]
    iota = lax.iota(jnp.int32, L)

    def start(ci, buf):
        pltpu.async_copy(
            table_h.at[idx_v.at[0, ci]], rows1_v.at[buf], sem1.at[buf])
        pltpu.async_copy(
            table_h.at[idx_v.at[1, ci]], rows2_v.at[buf], sem2.at[buf])

    def drain(buf):
        dummy = table_h.at[pl.ds(0, CH)]
        pltpu.make_async_copy(dummy, rows1_v.at[buf], sem1.at[buf]).wait()
        pltpu.make_async_copy(dummy, rows2_v.at[buf], sem2.at[buf]).wait()

    start(0, 0)
    start(1, 1)

    def chunk(ci, _):
        buf = lax.rem(ci, NBUF)

        @pl.when(ci + 2 < NCH)
        def _():
            start(ci + 2, lax.rem(ci + 2, NBUF))

        drain(buf)

        def group(g, _):
            base = g * L

            def sub(j, out16):
                r0 = base + j * 2
                for i in range(2):
                    r = r0 + i
                    acc = wv[0] * rows1_v[buf, r, pl.ds(0, L)]
                    for k in range(1, D // L):
                        acc = acc + wv[k] * rows1_v[buf, r, pl.ds(k * L, L)]
                    for k in range(D // L):
                        acc = acc + wv[D // L + k] * rows2_v[buf, r,
                                                            pl.ds(k * L, L)]
                    out16 = jnp.where(iota == j * 2 + i,
                                      out16 + jnp.sum(acc), out16)
                return out16

            out16 = lax.fori_loop(0, 8, sub, bv)
            y = 1.0 / (1.0 + jnp.exp(-out16))
            out_v[pl.ds(ci * CH + base, L)] = y
            return 0

        lax.fori_loop(0, CH // L, group, 0)
        return 0

    lax.fori_loop(0, NCH, chunk, 0)

    pltpu.sync_copy(out_v, out_h.at[pl.ds(wid * BPW, BPW)])


@jax.jit
def _run(table, idx, wb):
    mesh = plsc.VectorSubcoreMesh(core_axis_name="c", subcore_axis_name="s")
    k = functools.partial(
        pl.kernel,
        mesh=mesh,
        compiler_params=pltpu.CompilerParams(needs_layout_passes=False),
        out_type=jax.ShapeDtypeStruct((B,), jnp.float32),
        scratch_types=[
            pltpu.VMEM((2, NCH, CH), jnp.int32),
            pltpu.VMEM((NBUF, CH, D), jnp.float32),
            pltpu.VMEM((NBUF, CH, D), jnp.float32),
            pltpu.VMEM((2 * D + L,), jnp.float32),
            pltpu.VMEM((BPW,), jnp.float32),
            pltpu.SemaphoreType.DMA((NBUF,)),
            pltpu.SemaphoreType.DMA((NBUF,)),
        ],
    )(_body)
    return k(table, idx, wb)


def kernel(input1, input2, emb_table, fc_w, fc_b):
    idx = jnp.stack([input1.astype(jnp.int32), input2.astype(jnp.int32)])
    idx = idx.reshape(2, NW, NCH, CH).transpose(1, 0, 2, 3)
    wb = jnp.concatenate(
        [fc_w.reshape(2 * D),
         jnp.broadcast_to(fc_b.reshape(1), (L,)).astype(jnp.float32)])
    out = _run(emb_table, idx, wb)
    return out.reshape(B, 1)


# speedup vs baseline: 1.2274x; 1.0367x over previous
import functools

import jax
import jax.numpy as jnp
from jax import lax
from jax.experimental import pallas as pl
from jax.experimental.pallas import tpu as pltpu
from jax.experimental.pallas import tpu_sc as plsc

B = 16384
D = 128
NC = 2
NS = 16
NW = NC * NS
BPW = B // NW
CH = 128
NCH = BPW // CH
L = 16
NBUF = 2


def _body(table_h, idx1_h, idx2_h, wb_h, out_h,
          idx1_v, idx2_v, rows1_v, rows2_v, wb_v, out_v,
          sem1, sem2):
    c = lax.axis_index("c")
    s = lax.axis_index("s")
    wid = s * NC + c

    pltpu.sync_copy(idx1_h.at[wid], idx1_v)
    pltpu.sync_copy(idx2_h.at[wid], idx2_v)
    pltpu.sync_copy(wb_h, wb_v)

    wv = [wb_v[pl.ds(k * L, L)] for k in range(2 * D // L)]
    bv = wb_v[pl.ds(2 * D, L)]
    iota = lax.iota(jnp.int32, L)

    def start(ci, buf):
        pltpu.async_copy(
            table_h.at[idx1_v.at[ci]], rows1_v.at[buf], sem1.at[buf])
        pltpu.async_copy(
            table_h.at[idx2_v.at[ci]], rows2_v.at[buf], sem2.at[buf])

    def drain(buf):
        dummy = table_h.at[pl.ds(0, CH)]
        pltpu.make_async_copy(dummy, rows1_v.at[buf], sem1.at[buf]).wait()
        pltpu.make_async_copy(dummy, rows2_v.at[buf], sem2.at[buf]).wait()

    start(0, 0)

    def chunk(ci, _):
        buf = lax.rem(ci, NBUF)

        @pl.when(ci + 1 < NCH)
        def _():
            start(ci + 1, lax.rem(ci + 1, NBUF))

        drain(buf)

        def group(g, _):
            base = g * L

            def sub(j, out16):
                r0 = base + j * 2
                for i in range(2):
                    r = r0 + i
                    acc = wv[0] * rows1_v[buf, r, pl.ds(0, L)]
                    for k in range(1, D // L):
                        acc = acc + wv[k] * rows1_v[buf, r, pl.ds(k * L, L)]
                    for k in range(D // L):
                        acc = acc + wv[D // L + k] * rows2_v[buf, r,
                                                            pl.ds(k * L, L)]
                    out16 = jnp.where(iota == j * 2 + i,
                                      out16 + jnp.sum(acc), out16)
                return out16

            out16 = lax.fori_loop(0, 8, sub, bv)
            y = 1.0 / (1.0 + jnp.exp(-out16))
            out_v[pl.ds(ci * CH + base, L)] = y
            return 0

        lax.fori_loop(0, CH // L, group, 0)
        return 0

    lax.fori_loop(0, NCH, chunk, 0)

    pltpu.sync_copy(out_v, out_h.at[pl.ds(wid * BPW, BPW)])


@jax.jit
def _run(table, idx1, idx2, wb):
    mesh = plsc.VectorSubcoreMesh(core_axis_name="c", subcore_axis_name="s")
    k = functools.partial(
        pl.kernel,
        mesh=mesh,
        compiler_params=pltpu.CompilerParams(needs_layout_passes=False),
        out_type=jax.ShapeDtypeStruct((B,), jnp.float32),
        scratch_types=[
            pltpu.VMEM((NCH, CH), jnp.int32),
            pltpu.VMEM((NCH, CH), jnp.int32),
            pltpu.VMEM((NBUF, CH, D), jnp.float32),
            pltpu.VMEM((NBUF, CH, D), jnp.float32),
            pltpu.VMEM((2 * D + L,), jnp.float32),
            pltpu.VMEM((BPW,), jnp.float32),
            pltpu.SemaphoreType.DMA((NBUF,)),
            pltpu.SemaphoreType.DMA((NBUF,)),
        ],
    )(_body)
    return k(table, idx1, idx2, wb)


def kernel(input1, input2, emb_table, fc_w, fc_b):
    idx1 = input1.astype(jnp.int32).reshape(NW, NCH, CH)
    idx2 = input2.astype(jnp.int32).reshape(NW, NCH, CH)
    wb = jnp.concatenate(
        [fc_w.reshape(2 * D),
         jnp.broadcast_to(fc_b.reshape(1), (L,)).astype(jnp.float32)])
    out = _run(emb_table, idx1, idx2, wb)
    return out.reshape(B, 1)
